# baseline (speedup 1.0000x reference)
import jax
import jax.numpy as jnp
from jax.experimental import pallas as pl
from jax.experimental.pallas import tpu as pltpu

H = 2048; NH = 16; NKV = 16; HD = 128; E = 8; KTOP = 2
I = 1408; IS = 5632; EPS = 1e-6; THETA = 1000000.0; T = 2048

F32 = jnp.float32


def _k1_body(x_ref, ln_ref, w_ref, b_ref, cos_ref, sin_ref, out_ref, scale_ref):
    j = pl.program_id(0)

    @pl.when(j == 0)
    def _():
        xs = x_ref[...]
        scale_ref[...] = jax.lax.rsqrt(
            jnp.mean(xs * xs, axis=-1, keepdims=True) + EPS)

    xn = x_ref[...] * scale_ref[...] * ln_ref[...]
    acc = jax.lax.dot_general(xn, w_ref[...], (((1,), (1,)), ((), ())),
                              preferred_element_type=F32) + b_ref[...]
    c = cos_ref[...]
    s = sin_ref[...]
    x1 = acc[:, :HD // 2]
    x2 = acc[:, HD // 2:]
    roped = jnp.concatenate([x1 * c - x2 * s, x2 * c + x1 * s], axis=-1)
    out_ref[...] = jnp.where(j < 2 * NH, roped, acc)


def _qkv_rope(x, ln1_w, w_qkv, b_qkv, cos, sin):
    nblk = (NH + 2 * NKV)
    return pl.pallas_call(
        _k1_body,
        grid=(nblk,),
        in_specs=[
            pl.BlockSpec((T, H), lambda j: (0, 0)),
            pl.BlockSpec((1, H), lambda j: (0, 0)),
            pl.BlockSpec((HD, H), lambda j: (j, 0)),
            pl.BlockSpec((1, HD), lambda j: (0, j)),
            pl.BlockSpec((T, HD // 2), lambda j: (0, 0)),
            pl.BlockSpec((T, HD // 2), lambda j: (0, 0)),
        ],
        out_specs=pl.BlockSpec((T, HD), lambda j: (0, j)),
        out_shape=jax.ShapeDtypeStruct((T, nblk * HD), F32),
        scratch_shapes=[pltpu.VMEM((T, 1), F32)],
        compiler_params=pltpu.CompilerParams(
            dimension_semantics=("arbitrary",)),
    )(x, ln1_w.reshape(1, H), w_qkv, b_qkv.reshape(1, nblk * HD), cos, sin)


_BQ = 256
_BK = 256


def _k2_body(q_ref, k_ref, v_ref, o_ref):
    i = pl.program_id(1)
    s = jax.lax.dot_general(q_ref[...], k_ref[...], (((1,), (1,)), ((), ())),
                            preferred_element_type=F32) * (HD ** -0.5)
    rows = i * _BQ + jax.lax.broadcasted_iota(jnp.int32, (_BQ, T), 0)
    cols = jax.lax.broadcasted_iota(jnp.int32, (_BQ, T), 1)
    s = jnp.where(rows >= cols, s, -1e9)
    m = jnp.max(s, axis=-1, keepdims=True)
    p = jnp.exp(s - m)
    pn = p / jnp.sum(p, axis=-1, keepdims=True)
    o_ref[...] = jax.lax.dot_general(pn, v_ref[...], (((1,), (0,)), ((), ())),
                                     preferred_element_type=F32)

def _attention(qkv):
    return pl.pallas_call(
        _k2_body,
        grid=(NH, T // _BQ),
        in_specs=[
            pl.BlockSpec((_BQ, HD), lambda h, i: (i, h)),
            pl.BlockSpec((T, HD), lambda h, i: (0, NH + h)),
            pl.BlockSpec((T, HD), lambda h, i: (0, 2 * NH + h)),
        ],
        out_specs=pl.BlockSpec((_BQ, HD), lambda h, i: (i, h)),
        out_shape=jax.ShapeDtypeStruct((T, NH * HD), F32),
        compiler_params=pltpu.CompilerParams(
            dimension_semantics=("parallel", "arbitrary")),
    )(qkv, qkv, qkv)


_BM3 = 256


def _k3_body(x_ref, o_ref, wo_ref, ln2_ref, wr_ref,
             x1_ref, xn2_ref, logits_ref):
    x1 = x_ref[...] + jax.lax.dot_general(
        o_ref[...], wo_ref[...], (((1,), (1,)), ((), ())),
        preferred_element_type=F32)
    scale = jax.lax.rsqrt(jnp.mean(x1 * x1, axis=-1, keepdims=True) + EPS)
    xn2 = x1 * scale * ln2_ref[...]
    x1_ref[...] = x1
    xn2_ref[...] = xn2
    logits_ref[...] = jax.lax.dot_general(
        xn2, wr_ref[...], (((1,), (1,)), ((), ())),
        preferred_element_type=F32)


def _oproj_norm_router(x, o, w_o, ln2_w, wr):
    return pl.pallas_call(
        _k3_body,
        grid=(T // _BM3,),
        in_specs=[
            pl.BlockSpec((_BM3, H), lambda i: (i, 0)),
            pl.BlockSpec((_BM3, NH * HD), lambda i: (i, 0)),
            pl.BlockSpec((H, NH * HD), lambda i: (0, 0)),
            pl.BlockSpec((1, H), lambda i: (0, 0)),
            pl.BlockSpec((128, H), lambda i: (0, 0)),
        ],
        out_specs=[
            pl.BlockSpec((_BM3, H), lambda i: (i, 0)),
            pl.BlockSpec((_BM3, H), lambda i: (i, 0)),
            pl.BlockSpec((_BM3, 128), lambda i: (i, 0)),
        ],
        out_shape=[
            jax.ShapeDtypeStruct((T, H), F32),
            jax.ShapeDtypeStruct((T, H), F32),
            jax.ShapeDtypeStruct((T, 128), F32),
        ],
        compiler_params=pltpu.CompilerParams(
            dimension_semantics=("arbitrary",)),
    )(x, o, w_o, ln2_w.reshape(1, H), wr)


_BN4 = 128


def _k4_body(xn_ref, wg_ref, wu_ref, act_ref):
    xn = xn_ref[...]
    g = jax.lax.dot_general(xn, wg_ref[...], (((1,), (1,)), ((), ())),
                            preferred_element_type=F32)
    u = jax.lax.dot_general(xn, wu_ref[...], (((1,), (1,)), ((), ())),
                            preferred_element_type=F32)
    act_ref[...] = (g * jax.nn.sigmoid(g)) * u


def _shared_gateup(xn2, w_gu_shared):
    return pl.pallas_call(
        _k4_body,
        grid=(IS // _BN4,),
        in_specs=[
            pl.BlockSpec((T, H), lambda j: (0, 0)),
            pl.BlockSpec((_BN4, H), lambda j: (j, 0)),
            pl.BlockSpec((_BN4, H), lambda j: (IS // _BN4 + j, 0)),
        ],
        out_specs=pl.BlockSpec((T, _BN4), lambda j: (0, j)),
        out_shape=jax.ShapeDtypeStruct((T, IS), F32),
        compiler_params=pltpu.CompilerParams(
            dimension_semantics=("arbitrary",)),
    )(xn2, w_gu_shared, w_gu_shared)


_BI = 128


_BM = 128
NP = T * KTOP + E * _BM
NTILES = NP // _BM
_BIA = 128
_BIH = 256


def _kA_body(te_ref, xd_ref, wg_ref, wu_ref, act_ref):
    t = pl.program_id(1)
    xt = xd_ref[pl.ds(t * _BM, _BM), :]
    g = jax.lax.dot_general(xt, wg_ref[0], (((1,), (1,)), ((), ())),
                            preferred_element_type=F32)
    u = jax.lax.dot_general(xt, wu_ref[0], (((1,), (1,)), ((), ())),
                            preferred_element_type=F32)
    act_ref[...] = (g * jax.nn.sigmoid(g)) * u


def _moe_gateup(x_disp, w_gu_exp, tile_e):
    grid = (I // _BIA, NTILES)
    return pl.pallas_call(
        _kA_body,
        grid_spec=pltpu.PrefetchScalarGridSpec(
            num_scalar_prefetch=1,
            grid=grid,
            in_specs=[
                pl.BlockSpec((NP, H), lambda ia, t, te: (0, 0)),
                pl.BlockSpec((1, _BIA, H), lambda ia, t, te: (te[t], ia, 0)),
                pl.BlockSpec((1, _BIA, H),
                             lambda ia, t, te: (te[t], I // _BIA + ia, 0)),
            ],
            out_specs=pl.BlockSpec((_BM, _BIA), lambda ia, t, te: (t, ia)),
        ),
        out_shape=jax.ShapeDtypeStruct((NP, I), F32),
        compiler_params=pltpu.CompilerParams(
            dimension_semantics=("arbitrary", "arbitrary")),
    )(tile_e, x_disp, w_gu_exp, w_gu_exp)


def _kB_body(te_ref, act_ref, wd_ref, out_ref):
    t = pl.program_id(1)
    at = act_ref[pl.ds(t * _BM, _BM), :]
    out_ref[...] = jax.lax.dot_general(at, wd_ref[0], (((1,), (1,)), ((), ())),
                                       preferred_element_type=F32)


def _moe_down(act, w_down_exp, tile_e):
    grid = (H // _BIH, NTILES)
    return pl.pallas_call(
        _kB_body,
        grid_spec=pltpu.PrefetchScalarGridSpec(
            num_scalar_prefetch=1,
            grid=grid,
            in_specs=[
                pl.BlockSpec((NP, I), lambda h, t, te: (0, 0)),
                pl.BlockSpec((1, _BIH, I), lambda h, t, te: (te[t], h, 0)),
            ],
            out_specs=pl.BlockSpec((_BM, _BIH), lambda h, t, te: (t, h)),
        ),
        out_shape=jax.ShapeDtypeStruct((NP, H), F32),
        compiler_params=pltpu.CompilerParams(
            dimension_semantics=("arbitrary", "arbitrary")),
    )(tile_e, act, w_down_exp)


_BM5 = 256
_BN5 = 256


def _k5_body(act_ref, wd_ref, x1_ref, logits_ref, fused_ref, out_ref):
    shared = jax.lax.dot_general(act_ref[...], wd_ref[...],
                                 (((1,), (1,)), ((), ())),
                                 preferred_element_type=F32)
    sg = jax.nn.sigmoid(logits_ref[...][:, E:E + 1])
    out_ref[...] = x1_ref[...] + sg * shared + fused_ref[...]


def _shared_down_combine(act_s, w_down_shared, x1, logits, fused):
    return pl.pallas_call(
        _k5_body,
        grid=(T // _BM5, H // _BN5),
        in_specs=[
            pl.BlockSpec((_BM5, IS), lambda i, j: (i, 0)),
            pl.BlockSpec((_BN5, IS), lambda i, j: (j, 0)),
            pl.BlockSpec((_BM5, _BN5), lambda i, j: (i, j)),
            pl.BlockSpec((_BM5, 128), lambda i, j: (i, 0)),
            pl.BlockSpec((_BM5, _BN5), lambda i, j: (i, j)),
        ],
        out_specs=pl.BlockSpec((_BM5, _BN5), lambda i, j: (i, j)),
        out_shape=jax.ShapeDtypeStruct((T, H), F32),
        compiler_params=pltpu.CompilerParams(
            dimension_semantics=("parallel", "arbitrary")),
    )(act_s, w_down_shared, x1, logits, fused)


def kernel(positions, x, ln1_w, ln2_w, w_qkv, b_qkv, w_o, w_gate, w_sgate,
           w_gu_shared, w_down_shared, w_gu_exp, w_down_exp):
    half = HD // 2
    inv = THETA ** (-jnp.arange(half, dtype=F32) / half)
    f = positions.astype(F32)[:, None] * inv
    cos = jnp.cos(f)
    sin = jnp.sin(f)

    qkv = _qkv_rope(x, ln1_w, w_qkv, b_qkv, cos, sin)
    o = _attention(qkv)

    wr = jnp.concatenate(
        [w_gate, w_sgate, jnp.zeros((128 - E - 1, H), F32)], axis=0)
    x1, xn2, logits = _oproj_norm_router(x, o, w_o, ln2_w, wr)

    probs = jax.nn.softmax(logits[:, :E], axis=-1)
    vals, idx = jax.lax.top_k(probs, KTOP)
    vals = vals / jnp.sum(vals, axis=-1, keepdims=True)

    ex = idx.reshape(-1)
    order = jnp.argsort(ex)
    ex_sorted = ex[order]
    counts = jnp.bincount(ex, length=E)
    padded = ((counts + _BM - 1) // _BM) * _BM
    pstart = jnp.concatenate([jnp.zeros((1,), jnp.int32),
                              jnp.cumsum(padded)[:-1].astype(jnp.int32)])
    gstart = jnp.concatenate([jnp.zeros((1,), jnp.int32),
                              jnp.cumsum(counts)[:-1].astype(jnp.int32)])
    rank = jnp.arange(T * KTOP, dtype=jnp.int32) - gstart[ex_sorted]
    pos_sorted = pstart[ex_sorted] + rank
    tok_sorted = (order // KTOP).astype(jnp.int32)
    tok_for_pos = jnp.zeros((NP,), jnp.int32).at[pos_sorted].set(tok_sorted)
    pos_for_slot = jnp.zeros((T * KTOP,), jnp.int32).at[order].set(pos_sorted)
    pc = jnp.cumsum(padded)
    tile_e = jnp.minimum(
        jnp.searchsorted(pc, jnp.arange(NTILES, dtype=jnp.int32) * _BM,
                         side='right'),
        E - 1).astype(jnp.int32)

    x_disp = xn2[tok_for_pos]
    act = _moe_gateup(x_disp, w_gu_exp, tile_e)
    out_slot = _moe_down(act, w_down_exp, tile_e)

    pos_tk = pos_for_slot.reshape(T, KTOP)
    fused = (vals[:, :, None] * out_slot[pos_tk]).sum(axis=1)

    act_s = _shared_gateup(xn2, w_gu_shared)
    return _shared_down_combine(act_s, w_down_shared, x1, logits, fused)


# speedup vs baseline: 1.0529x; 1.0529x over previous
import jax
import jax.numpy as jnp
from jax.experimental import pallas as pl
from jax.experimental.pallas import tpu as pltpu

H = 2048; NH = 16; NKV = 16; HD = 128; E = 8; KTOP = 2
I = 1408; IS = 5632; EPS = 1e-6; THETA = 1000000.0; T = 2048

F32 = jnp.float32


_BW1 = 512


def _k1_body(x_ref, ln_ref, w_ref, b_ref, cos_ref, sin_ref, out_ref, scale_ref):
    j = pl.program_id(0)

    @pl.when(j == 0)
    def _():
        xs = x_ref[...]
        scale_ref[...] = jax.lax.rsqrt(
            jnp.mean(xs * xs, axis=-1, keepdims=True) + EPS)

    xn = x_ref[...] * scale_ref[...] * ln_ref[...]
    acc = jax.lax.dot_general(xn, w_ref[...], (((1,), (1,)), ((), ())),
                              preferred_element_type=F32) + b_ref[...]
    c = cos_ref[...]
    s = sin_ref[...]
    pieces = []
    for hloc in range(_BW1 // HD):
        x1 = acc[:, hloc * HD:hloc * HD + HD // 2]
        x2 = acc[:, hloc * HD + HD // 2:(hloc + 1) * HD]
        pieces.append(x1 * c - x2 * s)
        pieces.append(x2 * c + x1 * s)
    roped = jnp.concatenate(pieces, axis=-1)
    out_ref[...] = jnp.where(j < 2 * NH * HD // _BW1, roped, acc)


def _qkv_rope(x, ln1_w, w_qkv, b_qkv, cos, sin):
    nblk = (NH + 2 * NKV) * HD // _BW1
    return pl.pallas_call(
        _k1_body,
        grid=(nblk,),
        in_specs=[
            pl.BlockSpec((T, H), lambda j: (0, 0)),
            pl.BlockSpec((1, H), lambda j: (0, 0)),
            pl.BlockSpec((_BW1, H), lambda j: (j, 0)),
            pl.BlockSpec((1, _BW1), lambda j: (0, j)),
            pl.BlockSpec((T, HD // 2), lambda j: (0, 0)),
            pl.BlockSpec((T, HD // 2), lambda j: (0, 0)),
        ],
        out_specs=pl.BlockSpec((T, _BW1), lambda j: (0, j)),
        out_shape=jax.ShapeDtypeStruct((T, (NH + 2 * NKV) * HD), F32),
        scratch_shapes=[pltpu.VMEM((T, 1), F32)],
        compiler_params=pltpu.CompilerParams(
            dimension_semantics=("arbitrary",)),
    )(x, ln1_w.reshape(1, H), w_qkv, b_qkv.reshape(1, (NH + 2 * NKV) * HD),
      cos, sin)


_BQ = 256
_BK = 256


def _k2_body(q_ref, k_ref, v_ref, o_ref, s_scr):
    i = pl.program_id(1)
    q = q_ref[...]
    nc = i + 1

    def p1(c, m):
        kc = k_ref[pl.ds(c * _BK, _BK), :]
        s = jax.lax.dot_general(q, kc, (((1,), (1,)), ((), ())),
                                preferred_element_type=F32) * (HD ** -0.5)
        rows = i * _BQ + jax.lax.broadcasted_iota(jnp.int32, (_BQ, _BK), 0)
        cols = c * _BK + jax.lax.broadcasted_iota(jnp.int32, (_BQ, _BK), 1)
        s = jnp.where(rows >= cols, s, -1e9)
        s_scr[:, pl.ds(c * _BK, _BK)] = s
        return jnp.maximum(m, jnp.max(s, axis=-1, keepdims=True))

    m = jax.lax.fori_loop(0, nc, p1, jnp.full((_BQ, 1), -1e30, F32))

    def p2(c, l):
        p = jnp.exp(s_scr[:, pl.ds(c * _BK, _BK)] - m)
        s_scr[:, pl.ds(c * _BK, _BK)] = p
        return l + jnp.sum(p, axis=-1, keepdims=True)

    l = jax.lax.fori_loop(0, nc, p2, jnp.zeros((_BQ, 1), F32))

    def p3(c, acc):
        pn = s_scr[:, pl.ds(c * _BK, _BK)] / l
        vc = v_ref[pl.ds(c * _BK, _BK), :]
        return acc + jax.lax.dot_general(pn, vc, (((1,), (0,)), ((), ())),
                                         preferred_element_type=F32)

    o_ref[...] = jax.lax.fori_loop(0, nc, p3, jnp.zeros((_BQ, HD), F32))

def _attention(qkv):
    return pl.pallas_call(
        _k2_body,
        grid=(NH, T // _BQ),
        in_specs=[
            pl.BlockSpec((_BQ, HD), lambda h, i: (i, h)),
            pl.BlockSpec((T, HD), lambda h, i: (0, NH + h)),
            pl.BlockSpec((T, HD), lambda h, i: (0, 2 * NH + h)),
        ],
        out_specs=pl.BlockSpec((_BQ, HD), lambda h, i: (i, h)),
        out_shape=jax.ShapeDtypeStruct((T, NH * HD), F32),
        scratch_shapes=[pltpu.VMEM((_BQ, T), F32)],
        compiler_params=pltpu.CompilerParams(
            dimension_semantics=("parallel", "arbitrary")),
    )(qkv, qkv, qkv)


_BM3 = 256


def _k3_body(x_ref, o_ref, wo_ref, ln2_ref, wr_ref,
             x1_ref, xn2_ref, logits_ref):
    x1 = x_ref[...] + jax.lax.dot_general(
        o_ref[...], wo_ref[...], (((1,), (1,)), ((), ())),
        preferred_element_type=F32)
    scale = jax.lax.rsqrt(jnp.mean(x1 * x1, axis=-1, keepdims=True) + EPS)
    xn2 = x1 * scale * ln2_ref[...]
    x1_ref[...] = x1
    xn2_ref[...] = xn2
    logits_ref[...] = jax.lax.dot_general(
        xn2, wr_ref[...], (((1,), (1,)), ((), ())),
        preferred_element_type=F32)


def _oproj_norm_router(x, o, w_o, ln2_w, wr):
    return pl.pallas_call(
        _k3_body,
        grid=(T // _BM3,),
        in_specs=[
            pl.BlockSpec((_BM3, H), lambda i: (i, 0)),
            pl.BlockSpec((_BM3, NH * HD), lambda i: (i, 0)),
            pl.BlockSpec((H, NH * HD), lambda i: (0, 0)),
            pl.BlockSpec((1, H), lambda i: (0, 0)),
            pl.BlockSpec((128, H), lambda i: (0, 0)),
        ],
        out_specs=[
            pl.BlockSpec((_BM3, H), lambda i: (i, 0)),
            pl.BlockSpec((_BM3, H), lambda i: (i, 0)),
            pl.BlockSpec((_BM3, 128), lambda i: (i, 0)),
        ],
        out_shape=[
            jax.ShapeDtypeStruct((T, H), F32),
            jax.ShapeDtypeStruct((T, H), F32),
            jax.ShapeDtypeStruct((T, 128), F32),
        ],
        compiler_params=pltpu.CompilerParams(
            dimension_semantics=("arbitrary",)),
    )(x, o, w_o, ln2_w.reshape(1, H), wr)


_BN4 = 512


def _k4_body(xn_ref, wg_ref, wu_ref, act_ref):
    xn = xn_ref[...]
    g = jax.lax.dot_general(xn, wg_ref[...], (((1,), (1,)), ((), ())),
                            preferred_element_type=F32)
    u = jax.lax.dot_general(xn, wu_ref[...], (((1,), (1,)), ((), ())),
                            preferred_element_type=F32)
    act_ref[...] = (g * jax.nn.sigmoid(g)) * u


def _shared_gateup(xn2, w_gu_shared):
    return pl.pallas_call(
        _k4_body,
        grid=(IS // _BN4,),
        in_specs=[
            pl.BlockSpec((T, H), lambda j: (0, 0)),
            pl.BlockSpec((_BN4, H), lambda j: (j, 0)),
            pl.BlockSpec((_BN4, H), lambda j: (IS // _BN4 + j, 0)),
        ],
        out_specs=pl.BlockSpec((T, _BN4), lambda j: (0, j)),
        out_shape=jax.ShapeDtypeStruct((T, IS), F32),
        compiler_params=pltpu.CompilerParams(
            dimension_semantics=("arbitrary",)),
    )(xn2, w_gu_shared, w_gu_shared)


_BI = 128


_BM = 128
NP = T * KTOP + E * _BM
NTILES = NP // _BM
_BIA = 128
_BIH = 512


def _kA_body(te_ref, xd_ref, wg_ref, wu_ref, act_ref):
    t = pl.program_id(1)
    xt = xd_ref[pl.ds(t * _BM, _BM), :]
    g = jax.lax.dot_general(xt, wg_ref[0], (((1,), (1,)), ((), ())),
                            preferred_element_type=F32)
    u = jax.lax.dot_general(xt, wu_ref[0], (((1,), (1,)), ((), ())),
                            preferred_element_type=F32)
    act_ref[...] = (g * jax.nn.sigmoid(g)) * u


def _moe_gateup(x_disp, w_gu_exp, tile_e):
    grid = (I // _BIA, NTILES)
    return pl.pallas_call(
        _kA_body,
        grid_spec=pltpu.PrefetchScalarGridSpec(
            num_scalar_prefetch=1,
            grid=grid,
            in_specs=[
                pl.BlockSpec((NP, H), lambda ia, t, te: (0, 0)),
                pl.BlockSpec((1, _BIA, H), lambda ia, t, te: (te[t], ia, 0)),
                pl.BlockSpec((1, _BIA, H),
                             lambda ia, t, te: (te[t], I // _BIA + ia, 0)),
            ],
            out_specs=pl.BlockSpec((_BM, _BIA), lambda ia, t, te: (t, ia)),
        ),
        out_shape=jax.ShapeDtypeStruct((NP, I), F32),
        compiler_params=pltpu.CompilerParams(
            dimension_semantics=("arbitrary", "arbitrary")),
    )(tile_e, x_disp, w_gu_exp, w_gu_exp)


def _kB_body(te_ref, act_ref, wd_ref, out_ref):
    t = pl.program_id(1)
    at = act_ref[pl.ds(t * _BM, _BM), :]
    out_ref[...] = jax.lax.dot_general(at, wd_ref[0], (((1,), (1,)), ((), ())),
                                       preferred_element_type=F32)


def _moe_down(act, w_down_exp, tile_e):
    grid = (H // _BIH, NTILES)
    return pl.pallas_call(
        _kB_body,
        grid_spec=pltpu.PrefetchScalarGridSpec(
            num_scalar_prefetch=1,
            grid=grid,
            in_specs=[
                pl.BlockSpec((NP, I), lambda h, t, te: (0, 0)),
                pl.BlockSpec((1, _BIH, I), lambda h, t, te: (te[t], h, 0)),
            ],
            out_specs=pl.BlockSpec((_BM, _BIH), lambda h, t, te: (t, h)),
        ),
        out_shape=jax.ShapeDtypeStruct((NP, H), F32),
        compiler_params=pltpu.CompilerParams(
            dimension_semantics=("arbitrary", "arbitrary")),
    )(tile_e, act, w_down_exp)


_BM5 = 256
_BN5 = 256


def _k5_body(act_ref, wd_ref, x1_ref, logits_ref, fused_ref, out_ref):
    shared = jax.lax.dot_general(act_ref[...], wd_ref[...],
                                 (((1,), (1,)), ((), ())),
                                 preferred_element_type=F32)
    sg = jax.nn.sigmoid(logits_ref[...][:, E:E + 1])
    out_ref[...] = x1_ref[...] + sg * shared + fused_ref[...]


def _shared_down_combine(act_s, w_down_shared, x1, logits, fused):
    return pl.pallas_call(
        _k5_body,
        grid=(T // _BM5, H // _BN5),
        in_specs=[
            pl.BlockSpec((_BM5, IS), lambda i, j: (i, 0)),
            pl.BlockSpec((_BN5, IS), lambda i, j: (j, 0)),
            pl.BlockSpec((_BM5, _BN5), lambda i, j: (i, j)),
            pl.BlockSpec((_BM5, 128), lambda i, j: (i, 0)),
            pl.BlockSpec((_BM5, _BN5), lambda i, j: (i, j)),
        ],
        out_specs=pl.BlockSpec((_BM5, _BN5), lambda i, j: (i, j)),
        out_shape=jax.ShapeDtypeStruct((T, H), F32),
        compiler_params=pltpu.CompilerParams(
            dimension_semantics=("parallel", "arbitrary")),
    )(act_s, w_down_shared, x1, logits, fused)


def kernel(positions, x, ln1_w, ln2_w, w_qkv, b_qkv, w_o, w_gate, w_sgate,
           w_gu_shared, w_down_shared, w_gu_exp, w_down_exp):
    half = HD // 2
    inv = THETA ** (-jnp.arange(half, dtype=F32) / half)
    f = positions.astype(F32)[:, None] * inv
    cos = jnp.cos(f)
    sin = jnp.sin(f)

    qkv = _qkv_rope(x, ln1_w, w_qkv, b_qkv, cos, sin)
    o = _attention(qkv)

    wr = jnp.concatenate(
        [w_gate, w_sgate, jnp.zeros((128 - E - 1, H), F32)], axis=0)
    x1, xn2, logits = _oproj_norm_router(x, o, w_o, ln2_w, wr)

    probs = jax.nn.softmax(logits[:, :E], axis=-1)
    vals, idx = jax.lax.top_k(probs, KTOP)
    vals = vals / jnp.sum(vals, axis=-1, keepdims=True)

    ex = idx.reshape(-1)
    order = jnp.argsort(ex)
    ex_sorted = ex[order]
    counts = jnp.bincount(ex, length=E)
    padded = ((counts + _BM - 1) // _BM) * _BM
    pstart = jnp.concatenate([jnp.zeros((1,), jnp.int32),
                              jnp.cumsum(padded)[:-1].astype(jnp.int32)])
    gstart = jnp.concatenate([jnp.zeros((1,), jnp.int32),
                              jnp.cumsum(counts)[:-1].astype(jnp.int32)])
    rank = jnp.arange(T * KTOP, dtype=jnp.int32) - gstart[ex_sorted]
    pos_sorted = pstart[ex_sorted] + rank
    tok_sorted = (order // KTOP).astype(jnp.int32)
    tok_for_pos = jnp.zeros((NP,), jnp.int32).at[pos_sorted].set(tok_sorted)
    pos_for_slot = jnp.zeros((T * KTOP,), jnp.int32).at[order].set(pos_sorted)
    pc = jnp.cumsum(padded)
    tile_e = jnp.minimum(
        jnp.searchsorted(pc, jnp.arange(NTILES, dtype=jnp.int32) * _BM,
                         side='right'),
        E - 1).astype(jnp.int32)

    x_disp = xn2[tok_for_pos]
    act = _moe_gateup(x_disp, w_gu_exp, tile_e)
    out_slot = _moe_down(act, w_down_exp, tile_e)

    pos_tk = pos_for_slot.reshape(T, KTOP)
    fused = (vals[:, :, None] * out_slot[pos_tk]).sum(axis=1)

    act_s = _shared_gateup(xn2, w_gu_shared)
    return _shared_down_combine(act_s, w_down_shared, x1, logits, fused)


# speedup vs baseline: 1.2985x; 1.2332x over previous
import jax
import jax.numpy as jnp
from jax.experimental import pallas as pl
from jax.experimental.pallas import tpu as pltpu

H = 2048; NH = 16; NKV = 16; HD = 128; E = 8; KTOP = 2
I = 1408; IS = 5632; EPS = 1e-6; THETA = 1000000.0; T = 2048

F32 = jnp.float32


_BW1 = 512


def _k1_body(x_ref, ln_ref, w_ref, b_ref, cos_ref, sin_ref, out_ref, scale_ref):
    j = pl.program_id(0)

    @pl.when(j == 0)
    def _():
        xs = x_ref[...]
        scale_ref[...] = jax.lax.rsqrt(
            jnp.mean(xs * xs, axis=-1, keepdims=True) + EPS)

    xn = x_ref[...] * scale_ref[...] * ln_ref[...]
    acc = jax.lax.dot_general(xn, w_ref[...], (((1,), (1,)), ((), ())),
                              preferred_element_type=F32) + b_ref[...]
    c = cos_ref[...]
    s = sin_ref[...]
    pieces = []
    for hloc in range(_BW1 // HD):
        x1 = acc[:, hloc * HD:hloc * HD + HD // 2]
        x2 = acc[:, hloc * HD + HD // 2:(hloc + 1) * HD]
        pieces.append(x1 * c - x2 * s)
        pieces.append(x2 * c + x1 * s)
    roped = jnp.concatenate(pieces, axis=-1)
    out_ref[...] = jnp.where(j < 2 * NH * HD // _BW1, roped, acc)


def _qkv_rope(x, ln1_w, w_qkv, b_qkv, cos, sin):
    nblk = (NH + 2 * NKV) * HD // _BW1
    return pl.pallas_call(
        _k1_body,
        grid=(nblk,),
        in_specs=[
            pl.BlockSpec((T, H), lambda j: (0, 0)),
            pl.BlockSpec((1, H), lambda j: (0, 0)),
            pl.BlockSpec((_BW1, H), lambda j: (j, 0)),
            pl.BlockSpec((1, _BW1), lambda j: (0, j)),
            pl.BlockSpec((T, HD // 2), lambda j: (0, 0)),
            pl.BlockSpec((T, HD // 2), lambda j: (0, 0)),
        ],
        out_specs=pl.BlockSpec((T, _BW1), lambda j: (0, j)),
        out_shape=jax.ShapeDtypeStruct((T, (NH + 2 * NKV) * HD), F32),
        scratch_shapes=[pltpu.VMEM((T, 1), F32)],
        compiler_params=pltpu.CompilerParams(
            dimension_semantics=("arbitrary",)),
    )(x, ln1_w.reshape(1, H), w_qkv, b_qkv.reshape(1, (NH + 2 * NKV) * HD),
      cos, sin)


_BQ = 512
_BK = 512


def _k2_body(q_ref, k_ref, v_ref, o_ref, s_scr):
    i = pl.program_id(1)
    q = q_ref[...]
    nc = i + 1

    def p1(c, m):
        kc = k_ref[pl.ds(c * _BK, _BK), :]
        s = jax.lax.dot_general(q, kc, (((1,), (1,)), ((), ())),
                                preferred_element_type=F32) * (HD ** -0.5)
        rows = i * _BQ + jax.lax.broadcasted_iota(jnp.int32, (_BQ, _BK), 0)
        cols = c * _BK + jax.lax.broadcasted_iota(jnp.int32, (_BQ, _BK), 1)
        s = jnp.where(rows >= cols, s, -1e9)
        s_scr[:, pl.ds(c * _BK, _BK)] = s
        return jnp.maximum(m, jnp.max(s, axis=-1, keepdims=True))

    m = jax.lax.fori_loop(0, nc, p1, jnp.full((_BQ, 1), -1e30, F32))

    def p2(c, l):
        p = jnp.exp(s_scr[:, pl.ds(c * _BK, _BK)] - m)
        s_scr[:, pl.ds(c * _BK, _BK)] = p
        return l + jnp.sum(p, axis=-1, keepdims=True)

    l = jax.lax.fori_loop(0, nc, p2, jnp.zeros((_BQ, 1), F32))

    def p3(c, acc):
        pn = s_scr[:, pl.ds(c * _BK, _BK)] / l
        vc = v_ref[pl.ds(c * _BK, _BK), :]
        return acc + jax.lax.dot_general(pn, vc, (((1,), (0,)), ((), ())),
                                         preferred_element_type=F32)

    o_ref[...] = jax.lax.fori_loop(0, nc, p3, jnp.zeros((_BQ, HD), F32))

def _attention(qkv):
    return pl.pallas_call(
        _k2_body,
        grid=(NH, T // _BQ),
        in_specs=[
            pl.BlockSpec((_BQ, HD), lambda h, i: (i, h)),
            pl.BlockSpec((T, HD), lambda h, i: (0, NH + h)),
            pl.BlockSpec((T, HD), lambda h, i: (0, 2 * NH + h)),
        ],
        out_specs=pl.BlockSpec((_BQ, HD), lambda h, i: (i, h)),
        out_shape=jax.ShapeDtypeStruct((T, NH * HD), F32),
        scratch_shapes=[pltpu.VMEM((_BQ, T), F32)],
        compiler_params=pltpu.CompilerParams(
            dimension_semantics=("parallel", "arbitrary")),
    )(qkv, qkv, qkv)


_BM3 = 256


def _k3_body(x_ref, o_ref, wo_ref, ln2_ref, wr_ref,
             x1_ref, xn2_ref, logits_ref):
    x1 = x_ref[...] + jax.lax.dot_general(
        o_ref[...], wo_ref[...], (((1,), (1,)), ((), ())),
        preferred_element_type=F32)
    scale = jax.lax.rsqrt(jnp.mean(x1 * x1, axis=-1, keepdims=True) + EPS)
    xn2 = x1 * scale * ln2_ref[...]
    x1_ref[...] = x1
    xn2_ref[...] = xn2
    logits_ref[...] = jax.lax.dot_general(
        xn2, wr_ref[...], (((1,), (1,)), ((), ())),
        preferred_element_type=F32)


def _oproj_norm_router(x, o, w_o, ln2_w, wr):
    return pl.pallas_call(
        _k3_body,
        grid=(T // _BM3,),
        in_specs=[
            pl.BlockSpec((_BM3, H), lambda i: (i, 0)),
            pl.BlockSpec((_BM3, NH * HD), lambda i: (i, 0)),
            pl.BlockSpec((H, NH * HD), lambda i: (0, 0)),
            pl.BlockSpec((1, H), lambda i: (0, 0)),
            pl.BlockSpec((128, H), lambda i: (0, 0)),
        ],
        out_specs=[
            pl.BlockSpec((_BM3, H), lambda i: (i, 0)),
            pl.BlockSpec((_BM3, H), lambda i: (i, 0)),
            pl.BlockSpec((_BM3, 128), lambda i: (i, 0)),
        ],
        out_shape=[
            jax.ShapeDtypeStruct((T, H), F32),
            jax.ShapeDtypeStruct((T, H), F32),
            jax.ShapeDtypeStruct((T, 128), F32),
        ],
        compiler_params=pltpu.CompilerParams(
            dimension_semantics=("arbitrary",)),
    )(x, o, w_o, ln2_w.reshape(1, H), wr)


_BN4 = 512


def _k4_body(xn_ref, wg_ref, wu_ref, act_ref):
    xn = xn_ref[...]
    g = jax.lax.dot_general(xn, wg_ref[...], (((1,), (1,)), ((), ())),
                            preferred_element_type=F32)
    u = jax.lax.dot_general(xn, wu_ref[...], (((1,), (1,)), ((), ())),
                            preferred_element_type=F32)
    act_ref[...] = (g * jax.nn.sigmoid(g)) * u


def _shared_gateup(xn2, w_gu_shared):
    return pl.pallas_call(
        _k4_body,
        grid=(IS // _BN4,),
        in_specs=[
            pl.BlockSpec((T, H), lambda j: (0, 0)),
            pl.BlockSpec((_BN4, H), lambda j: (j, 0)),
            pl.BlockSpec((_BN4, H), lambda j: (IS // _BN4 + j, 0)),
        ],
        out_specs=pl.BlockSpec((T, _BN4), lambda j: (0, j)),
        out_shape=jax.ShapeDtypeStruct((T, IS), F32),
        compiler_params=pltpu.CompilerParams(
            dimension_semantics=("arbitrary",)),
    )(xn2, w_gu_shared, w_gu_shared)


_BI = 128


_BM = 256
NP = T * KTOP + E * _BM
NTILES = NP // _BM
_BIA = 128
_BIH = 512


def _kA_body(te_ref, xd_ref, wg_ref, wu_ref, act_ref):
    t = pl.program_id(1)
    xt = xd_ref[pl.ds(t * _BM, _BM), :]
    g = jax.lax.dot_general(xt, wg_ref[0], (((1,), (1,)), ((), ())),
                            preferred_element_type=F32)
    u = jax.lax.dot_general(xt, wu_ref[0], (((1,), (1,)), ((), ())),
                            preferred_element_type=F32)
    act_ref[...] = (g * jax.nn.sigmoid(g)) * u


def _moe_gateup(x_disp, w_gu_exp, tile_e):
    grid = (I // _BIA, NTILES)
    return pl.pallas_call(
        _kA_body,
        grid_spec=pltpu.PrefetchScalarGridSpec(
            num_scalar_prefetch=1,
            grid=grid,
            in_specs=[
                pl.BlockSpec((NP, H), lambda ia, t, te: (0, 0)),
                pl.BlockSpec((1, _BIA, H), lambda ia, t, te: (te[t], ia, 0)),
                pl.BlockSpec((1, _BIA, H),
                             lambda ia, t, te: (te[t], I // _BIA + ia, 0)),
            ],
            out_specs=pl.BlockSpec((_BM, _BIA), lambda ia, t, te: (t, ia)),
        ),
        out_shape=jax.ShapeDtypeStruct((NP, I), F32),
        compiler_params=pltpu.CompilerParams(
            dimension_semantics=("arbitrary", "arbitrary")),
    )(tile_e, x_disp, w_gu_exp, w_gu_exp)


def _kB_body(te_ref, act_ref, wd_ref, out_ref):
    t = pl.program_id(1)
    at = act_ref[pl.ds(t * _BM, _BM), :]
    out_ref[...] = jax.lax.dot_general(at, wd_ref[0], (((1,), (1,)), ((), ())),
                                       preferred_element_type=F32)


def _moe_down(act, w_down_exp, tile_e):
    grid = (H // _BIH, NTILES)
    return pl.pallas_call(
        _kB_body,
        grid_spec=pltpu.PrefetchScalarGridSpec(
            num_scalar_prefetch=1,
            grid=grid,
            in_specs=[
                pl.BlockSpec((NP, I), lambda h, t, te: (0, 0)),
                pl.BlockSpec((1, _BIH, I), lambda h, t, te: (te[t], h, 0)),
            ],
            out_specs=pl.BlockSpec((_BM, _BIH), lambda h, t, te: (t, h)),
        ),
        out_shape=jax.ShapeDtypeStruct((NP, H), F32),
        compiler_params=pltpu.CompilerParams(
            dimension_semantics=("arbitrary", "arbitrary")),
    )(tile_e, act, w_down_exp)


_BM5 = 256
_BN5 = 256


def _k5_body(act_ref, wd_ref, x1_ref, logits_ref, fused_ref, out_ref):
    shared = jax.lax.dot_general(act_ref[...], wd_ref[...],
                                 (((1,), (1,)), ((), ())),
                                 preferred_element_type=F32)
    sg = jax.nn.sigmoid(logits_ref[...][:, E:E + 1])
    out_ref[...] = x1_ref[...] + sg * shared + fused_ref[...]


def _shared_down_combine(act_s, w_down_shared, x1, logits, fused):
    return pl.pallas_call(
        _k5_body,
        grid=(T // _BM5, H // _BN5),
        in_specs=[
            pl.BlockSpec((_BM5, IS), lambda i, j: (i, 0)),
            pl.BlockSpec((_BN5, IS), lambda i, j: (j, 0)),
            pl.BlockSpec((_BM5, _BN5), lambda i, j: (i, j)),
            pl.BlockSpec((_BM5, 128), lambda i, j: (i, 0)),
            pl.BlockSpec((_BM5, _BN5), lambda i, j: (i, j)),
        ],
        out_specs=pl.BlockSpec((_BM5, _BN5), lambda i, j: (i, j)),
        out_shape=jax.ShapeDtypeStruct((T, H), F32),
        compiler_params=pltpu.CompilerParams(
            dimension_semantics=("parallel", "arbitrary")),
    )(act_s, w_down_shared, x1, logits, fused)


def kernel(positions, x, ln1_w, ln2_w, w_qkv, b_qkv, w_o, w_gate, w_sgate,
           w_gu_shared, w_down_shared, w_gu_exp, w_down_exp):
    half = HD // 2
    inv = THETA ** (-jnp.arange(half, dtype=F32) / half)
    f = positions.astype(F32)[:, None] * inv
    cos = jnp.cos(f)
    sin = jnp.sin(f)

    qkv = _qkv_rope(x, ln1_w, w_qkv, b_qkv, cos, sin)
    o = _attention(qkv)

    wr = jnp.concatenate(
        [w_gate, w_sgate, jnp.zeros((128 - E - 1, H), F32)], axis=0)
    x1, xn2, logits = _oproj_norm_router(x, o, w_o, ln2_w, wr)

    probs = jax.nn.softmax(logits[:, :E], axis=-1)
    vals, idx = jax.lax.top_k(probs, KTOP)
    vals = vals / jnp.sum(vals, axis=-1, keepdims=True)

    ex = idx.reshape(-1)
    order = jnp.argsort(ex)
    ex_sorted = ex[order]
    counts = jnp.bincount(ex, length=E)
    padded = ((counts + _BM - 1) // _BM) * _BM
    pstart = jnp.concatenate([jnp.zeros((1,), jnp.int32),
                              jnp.cumsum(padded)[:-1].astype(jnp.int32)])
    gstart = jnp.concatenate([jnp.zeros((1,), jnp.int32),
                              jnp.cumsum(counts)[:-1].astype(jnp.int32)])
    rank = jnp.arange(T * KTOP, dtype=jnp.int32) - gstart[ex_sorted]
    pos_sorted = pstart[ex_sorted] + rank
    tok_sorted = (order // KTOP).astype(jnp.int32)
    tok_for_pos = jnp.zeros((NP,), jnp.int32).at[pos_sorted].set(tok_sorted)
    pos_for_slot = jnp.zeros((T * KTOP,), jnp.int32).at[order].set(pos_sorted)
    pc = jnp.cumsum(padded)
    tile_e = jnp.minimum(
        jnp.searchsorted(pc, jnp.arange(NTILES, dtype=jnp.int32) * _BM,
                         side='right'),
        E - 1).astype(jnp.int32)

    x_disp = xn2[tok_for_pos]
    act = _moe_gateup(x_disp, w_gu_exp, tile_e)
    out_slot = _moe_down(act, w_down_exp, tile_e)

    pos_tk = pos_for_slot.reshape(T, KTOP)
    fused = (vals[:, :, None] * out_slot[pos_tk]).sum(axis=1)

    act_s = _shared_gateup(xn2, w_gu_shared)
    return _shared_down_combine(act_s, w_down_shared, x1, logits, fused)


# speedup vs baseline: 1.4165x; 1.0909x over previous
import jax
import jax.numpy as jnp
from jax.experimental import pallas as pl
from jax.experimental.pallas import tpu as pltpu

H = 2048; NH = 16; NKV = 16; HD = 128; E = 8; KTOP = 2
I = 1408; IS = 5632; EPS = 1e-6; THETA = 1000000.0; T = 2048

F32 = jnp.float32


_BW1 = 512


def _k1_body(x_ref, ln_ref, w_ref, b_ref, cos_ref, sin_ref, out_ref, scale_ref):
    j = pl.program_id(0)

    @pl.when(j == 0)
    def _():
        xs = x_ref[...]
        scale_ref[...] = jax.lax.rsqrt(
            jnp.mean(xs * xs, axis=-1, keepdims=True) + EPS)

    xn = x_ref[...] * scale_ref[...] * ln_ref[...]
    acc = jax.lax.dot_general(xn, w_ref[...], (((1,), (1,)), ((), ())),
                              preferred_element_type=F32) + b_ref[...]
    c = cos_ref[...]
    s = sin_ref[...]
    pieces = []
    for hloc in range(_BW1 // HD):
        x1 = acc[:, hloc * HD:hloc * HD + HD // 2]
        x2 = acc[:, hloc * HD + HD // 2:(hloc + 1) * HD]
        pieces.append(x1 * c - x2 * s)
        pieces.append(x2 * c + x1 * s)
    roped = jnp.concatenate(pieces, axis=-1)
    out_ref[...] = jnp.where(j < 2 * NH * HD // _BW1, roped, acc)


def _qkv_rope(x, ln1_w, w_qkv, b_qkv, cos, sin):
    nblk = (NH + 2 * NKV) * HD // _BW1
    return pl.pallas_call(
        _k1_body,
        grid=(nblk,),
        in_specs=[
            pl.BlockSpec((T, H), lambda j: (0, 0)),
            pl.BlockSpec((1, H), lambda j: (0, 0)),
            pl.BlockSpec((_BW1, H), lambda j: (j, 0)),
            pl.BlockSpec((1, _BW1), lambda j: (0, j)),
            pl.BlockSpec((T, HD // 2), lambda j: (0, 0)),
            pl.BlockSpec((T, HD // 2), lambda j: (0, 0)),
        ],
        out_specs=pl.BlockSpec((T, _BW1), lambda j: (0, j)),
        out_shape=jax.ShapeDtypeStruct((T, (NH + 2 * NKV) * HD), F32),
        scratch_shapes=[pltpu.VMEM((T, 1), F32)],
        compiler_params=pltpu.CompilerParams(
            dimension_semantics=("arbitrary",)),
    )(x, ln1_w.reshape(1, H), w_qkv, b_qkv.reshape(1, (NH + 2 * NKV) * HD),
      cos, sin)


_BQ = 512
_BK = 512


def _k2_body(q_ref, k_ref, v_ref, o_ref, s_scr):
    i = pl.program_id(1)
    q = q_ref[...]
    nc = i + 1

    def p1(c, m):
        kc = k_ref[pl.ds(c * _BK, _BK), :]
        s = jax.lax.dot_general(q, kc, (((1,), (1,)), ((), ())),
                                preferred_element_type=F32) * (HD ** -0.5)
        rows = i * _BQ + jax.lax.broadcasted_iota(jnp.int32, (_BQ, _BK), 0)
        cols = c * _BK + jax.lax.broadcasted_iota(jnp.int32, (_BQ, _BK), 1)
        s = jnp.where(rows >= cols, s, -1e9)
        s_scr[:, pl.ds(c * _BK, _BK)] = s
        return jnp.maximum(m, jnp.max(s, axis=-1, keepdims=True))

    m = jax.lax.fori_loop(0, nc, p1, jnp.full((_BQ, 1), -1e30, F32))

    def p2(c, l):
        p = jnp.exp(s_scr[:, pl.ds(c * _BK, _BK)] - m)
        s_scr[:, pl.ds(c * _BK, _BK)] = p
        return l + jnp.sum(p, axis=-1, keepdims=True)

    l = jax.lax.fori_loop(0, nc, p2, jnp.zeros((_BQ, 1), F32))

    def p3(c, acc):
        pn = s_scr[:, pl.ds(c * _BK, _BK)] / l
        vc = v_ref[pl.ds(c * _BK, _BK), :]
        return acc + jax.lax.dot_general(pn, vc, (((1,), (0,)), ((), ())),
                                         preferred_element_type=F32)

    o_ref[...] = jax.lax.fori_loop(0, nc, p3, jnp.zeros((_BQ, HD), F32))

def _attention(qkv):
    return pl.pallas_call(
        _k2_body,
        grid=(NH, T // _BQ),
        in_specs=[
            pl.BlockSpec((_BQ, HD), lambda h, i: (i, h)),
            pl.BlockSpec((T, HD), lambda h, i: (0, NH + h)),
            pl.BlockSpec((T, HD), lambda h, i: (0, 2 * NH + h)),
        ],
        out_specs=pl.BlockSpec((_BQ, HD), lambda h, i: (i, h)),
        out_shape=jax.ShapeDtypeStruct((T, NH * HD), F32),
        scratch_shapes=[pltpu.VMEM((_BQ, T), F32)],
        compiler_params=pltpu.CompilerParams(
            dimension_semantics=("parallel", "arbitrary")),
    )(qkv, qkv, qkv)


_BM3 = 256


def _k3_body(x_ref, o_ref, wo_ref, ln2_ref, wr_ref,
             x1_ref, xn2_ref, logits_ref):
    x1 = x_ref[...] + jax.lax.dot_general(
        o_ref[...], wo_ref[...], (((1,), (1,)), ((), ())),
        preferred_element_type=F32)
    scale = jax.lax.rsqrt(jnp.mean(x1 * x1, axis=-1, keepdims=True) + EPS)
    xn2 = x1 * scale * ln2_ref[...]
    x1_ref[...] = x1
    xn2_ref[...] = xn2
    logits_ref[...] = jax.lax.dot_general(
        xn2, wr_ref[...], (((1,), (1,)), ((), ())),
        preferred_element_type=F32)


def _oproj_norm_router(x, o, w_o, ln2_w, wr):
    return pl.pallas_call(
        _k3_body,
        grid=(T // _BM3,),
        in_specs=[
            pl.BlockSpec((_BM3, H), lambda i: (i, 0)),
            pl.BlockSpec((_BM3, NH * HD), lambda i: (i, 0)),
            pl.BlockSpec((H, NH * HD), lambda i: (0, 0)),
            pl.BlockSpec((1, H), lambda i: (0, 0)),
            pl.BlockSpec((128, H), lambda i: (0, 0)),
        ],
        out_specs=[
            pl.BlockSpec((_BM3, H), lambda i: (i, 0)),
            pl.BlockSpec((_BM3, H), lambda i: (i, 0)),
            pl.BlockSpec((_BM3, 128), lambda i: (i, 0)),
        ],
        out_shape=[
            jax.ShapeDtypeStruct((T, H), F32),
            jax.ShapeDtypeStruct((T, H), F32),
            jax.ShapeDtypeStruct((T, 128), F32),
        ],
        compiler_params=pltpu.CompilerParams(
            dimension_semantics=("arbitrary",)),
    )(x, o, w_o, ln2_w.reshape(1, H), wr)


_BN4 = 512


def _k4_body(xn_ref, wg_ref, wu_ref, act_ref):
    xn = xn_ref[...]
    g = jax.lax.dot_general(xn, wg_ref[...], (((1,), (1,)), ((), ())),
                            preferred_element_type=F32)
    u = jax.lax.dot_general(xn, wu_ref[...], (((1,), (1,)), ((), ())),
                            preferred_element_type=F32)
    act_ref[...] = (g * jax.nn.sigmoid(g)) * u


def _shared_gateup(xn2, w_gu_shared):
    return pl.pallas_call(
        _k4_body,
        grid=(IS // _BN4,),
        in_specs=[
            pl.BlockSpec((T, H), lambda j: (0, 0)),
            pl.BlockSpec((_BN4, H), lambda j: (j, 0)),
            pl.BlockSpec((_BN4, H), lambda j: (IS // _BN4 + j, 0)),
        ],
        out_specs=pl.BlockSpec((T, _BN4), lambda j: (0, j)),
        out_shape=jax.ShapeDtypeStruct((T, IS), F32),
        compiler_params=pltpu.CompilerParams(
            dimension_semantics=("arbitrary",)),
    )(xn2, w_gu_shared, w_gu_shared)


_BI = 128


_BM = 256
NP = T * KTOP + E * _BM
NTILES = NP // _BM
_BIA = 128
_BIH = 1024


def _kA_body(te_ref, xd_ref, wg_ref, wu_ref, act_ref):
    t = pl.program_id(1)
    xt = xd_ref[pl.ds(t * _BM, _BM), :]
    g = jax.lax.dot_general(xt, wg_ref[0], (((1,), (1,)), ((), ())),
                            preferred_element_type=F32)
    u = jax.lax.dot_general(xt, wu_ref[0], (((1,), (1,)), ((), ())),
                            preferred_element_type=F32)
    act_ref[...] = (g * jax.nn.sigmoid(g)) * u


def _moe_gateup(x_disp, w_gu_exp, tile_e):
    grid = (I // _BIA, NTILES)
    return pl.pallas_call(
        _kA_body,
        grid_spec=pltpu.PrefetchScalarGridSpec(
            num_scalar_prefetch=1,
            grid=grid,
            in_specs=[
                pl.BlockSpec((NP, H), lambda ia, t, te: (0, 0)),
                pl.BlockSpec((1, _BIA, H), lambda ia, t, te: (te[t], ia, 0)),
                pl.BlockSpec((1, _BIA, H),
                             lambda ia, t, te: (te[t], I // _BIA + ia, 0)),
            ],
            out_specs=pl.BlockSpec((_BM, _BIA), lambda ia, t, te: (t, ia)),
        ),
        out_shape=jax.ShapeDtypeStruct((NP, I), F32),
        compiler_params=pltpu.CompilerParams(
            dimension_semantics=("arbitrary", "arbitrary")),
    )(tile_e, x_disp, w_gu_exp, w_gu_exp)


def _kB_body(te_ref, act_ref, wd_ref, out_ref):
    t = pl.program_id(1)
    at = act_ref[pl.ds(t * _BM, _BM), :]
    out_ref[...] = jax.lax.dot_general(at, wd_ref[0], (((1,), (1,)), ((), ())),
                                       preferred_element_type=F32)


def _moe_down(act, w_down_exp, tile_e):
    grid = (H // _BIH, NTILES)
    return pl.pallas_call(
        _kB_body,
        grid_spec=pltpu.PrefetchScalarGridSpec(
            num_scalar_prefetch=1,
            grid=grid,
            in_specs=[
                pl.BlockSpec((NP, I), lambda h, t, te: (0, 0)),
                pl.BlockSpec((1, _BIH, I), lambda h, t, te: (te[t], h, 0)),
            ],
            out_specs=pl.BlockSpec((_BM, _BIH), lambda h, t, te: (t, h)),
        ),
        out_shape=jax.ShapeDtypeStruct((NP, H), F32),
        compiler_params=pltpu.CompilerParams(
            dimension_semantics=("arbitrary", "arbitrary")),
    )(tile_e, act, w_down_exp)


_BM5 = 256
_BN5 = 256


def _k5_body(act_ref, wd_ref, x1_ref, logits_ref, fused_ref, out_ref):
    shared = jax.lax.dot_general(act_ref[...], wd_ref[...],
                                 (((1,), (1,)), ((), ())),
                                 preferred_element_type=F32)
    sg = jax.nn.sigmoid(logits_ref[...][:, E:E + 1])
    out_ref[...] = x1_ref[...] + sg * shared + fused_ref[...]


def _shared_down_combine(act_s, w_down_shared, x1, logits, fused):
    return pl.pallas_call(
        _k5_body,
        grid=(T // _BM5, H // _BN5),
        in_specs=[
            pl.BlockSpec((_BM5, IS), lambda i, j: (i, 0)),
            pl.BlockSpec((_BN5, IS), lambda i, j: (j, 0)),
            pl.BlockSpec((_BM5, _BN5), lambda i, j: (i, j)),
            pl.BlockSpec((_BM5, 128), lambda i, j: (i, 0)),
            pl.BlockSpec((_BM5, _BN5), lambda i, j: (i, j)),
        ],
        out_specs=pl.BlockSpec((_BM5, _BN5), lambda i, j: (i, j)),
        out_shape=jax.ShapeDtypeStruct((T, H), F32),
        compiler_params=pltpu.CompilerParams(
            dimension_semantics=("parallel", "arbitrary")),
    )(act_s, w_down_shared, x1, logits, fused)


def kernel(positions, x, ln1_w, ln2_w, w_qkv, b_qkv, w_o, w_gate, w_sgate,
           w_gu_shared, w_down_shared, w_gu_exp, w_down_exp):
    half = HD // 2
    inv = THETA ** (-jnp.arange(half, dtype=F32) / half)
    f = positions.astype(F32)[:, None] * inv
    cos = jnp.cos(f)
    sin = jnp.sin(f)

    qkv = _qkv_rope(x, ln1_w, w_qkv, b_qkv, cos, sin)
    o = _attention(qkv)

    wr = jnp.concatenate(
        [w_gate, w_sgate, jnp.zeros((128 - E - 1, H), F32)], axis=0)
    x1, xn2, logits = _oproj_norm_router(x, o, w_o, ln2_w, wr)

    probs = jax.nn.softmax(logits[:, :E], axis=-1)
    vals, idx = jax.lax.top_k(probs, KTOP)
    vals = vals / jnp.sum(vals, axis=-1, keepdims=True)

    ex = idx.reshape(-1)
    order = jnp.argsort(ex)
    ex_sorted = ex[order]
    counts = jnp.bincount(ex, length=E)
    padded = ((counts + _BM - 1) // _BM) * _BM
    pstart = jnp.concatenate([jnp.zeros((1,), jnp.int32),
                              jnp.cumsum(padded)[:-1].astype(jnp.int32)])
    gstart = jnp.concatenate([jnp.zeros((1,), jnp.int32),
                              jnp.cumsum(counts)[:-1].astype(jnp.int32)])
    rank = jnp.arange(T * KTOP, dtype=jnp.int32) - gstart[ex_sorted]
    pos_sorted = pstart[ex_sorted] + rank
    tok_sorted = (order // KTOP).astype(jnp.int32)
    tok_for_pos = jnp.zeros((NP,), jnp.int32).at[pos_sorted].set(tok_sorted)
    pos_for_slot = jnp.zeros((T * KTOP,), jnp.int32).at[order].set(pos_sorted)
    pc = jnp.cumsum(padded)
    tile_e = jnp.minimum(
        jnp.searchsorted(pc, jnp.arange(NTILES, dtype=jnp.int32) * _BM,
                         side='right'),
        E - 1).astype(jnp.int32)

    x_disp = xn2[tok_for_pos]
    act = _moe_gateup(x_disp, w_gu_exp, tile_e)
    out_slot = _moe_down(act, w_down_exp, tile_e)

    pos_tk = pos_for_slot.reshape(T, KTOP)
    fused = (vals[:, 0:1] * out_slot[pos_tk[:, 0]]
             + vals[:, 1:2] * out_slot[pos_tk[:, 1]])

    act_s = _shared_gateup(xn2, w_gu_shared)
    return _shared_down_combine(act_s, w_down_shared, x1, logits, fused)
